# Initial kernel scaffold; baseline (speedup 1.0000x reference)
import jax
import jax.numpy as jnp
from jax.experimental import pallas as pl


def kernel(query, relation_weight, eos_weight, w_ih, w_hh, b_ih, b_hh):
    raise NotImplementedError("write your pallas kernel here")


# speedup vs baseline: 2.1964x; 2.1964x over previous
import jax
import jax.numpy as jnp
from jax.experimental import pallas as pl

_D = 256
_K = 1024
_HOPS = 8
_BLK = 512


def _mm_t(a, b):
    return jax.lax.dot_general(
        a, b, (((1,), (1,)), ((), ())),
        preferred_element_type=jnp.float32)


def _rulernn_body(q_ref, rw_ref, eos_ref, wih_ref, whh_ref, bih_ref, bhh_ref,
                  out_ref):
    f32 = jnp.float32
    q = q_ref[...]
    rw = rw_ref[...]
    wih = wih_ref[...]
    whh = whh_ref[...]
    bih = bih_ref[...]
    bhh = bhh_ref[...]

    wsq = jax.lax.dot_general(
        jnp.ones((1, _D), f32), rw * rw, (((1,), (1,)), ((), ())),
        preferred_element_type=f32, precision=jax.lax.Precision.HIGHEST)

    def gru(gi, gh, h):
        r = jax.nn.sigmoid(gi[:, :_D] + gh[:, :_D])
        z = jax.nn.sigmoid(gi[:, _D:2 * _D] + gh[:, _D:2 * _D])
        n = jnp.tanh(gi[:, 2 * _D:] + r * gh[:, 2 * _D:])
        return (1.0 - z) * n + z * h

    gi = _mm_t(q, wih) + bih
    r = jax.nn.sigmoid(gi[:, :_D] + bhh[:, :_D])
    z = jax.nn.sigmoid(gi[:, _D:2 * _D] + bhh[:, _D:2 * _D])
    n = jnp.tanh(gi[:, 2 * _D:] + r * bhh[:, 2 * _D:])
    h = (1.0 - z) * n

    gi2 = _mm_t(eos_ref[...], wih) + bih
    gh = _mm_t(h, whh) + bhh
    h = gru(gi2, gh, h)

    iota_k = jax.lax.broadcasted_iota(jnp.int32, (q.shape[0], _K), 1)
    for hop in range(_HOPS):
        hsq = jnp.sum(h * h, axis=1, keepdims=True)
        dist = hsq - 2.0 * _mm_t(h, rw) + wsq
        m = jnp.min(dist, axis=1, keepdims=True)
        idx = jnp.min(jnp.where(dist == m, iota_k, _K), axis=1, keepdims=True)
        onehot = (iota_k == idx).astype(f32)
        sg = jnp.dot(onehot, rw, preferred_element_type=f32)
        out_ref[:, hop, :] = sg
        gi = _mm_t(sg, wih) + bih
        gh = _mm_t(h, whh) + bhh
        h = gru(gi, gh, h)


def kernel(query, relation_weight, eos_weight, w_ih, w_hh, b_ih, b_hh):
    B, d = query.shape
    subgoals = pl.pallas_call(
        _rulernn_body,
        grid=(B // _BLK,),
        in_specs=[
            pl.BlockSpec((_BLK, d), lambda i: (i, 0)),
            pl.BlockSpec((_K, d), lambda i: (0, 0)),
            pl.BlockSpec((1, d), lambda i: (0, 0)),
            pl.BlockSpec((3 * d, d), lambda i: (0, 0)),
            pl.BlockSpec((3 * d, d), lambda i: (0, 0)),
            pl.BlockSpec((1, 3 * d), lambda i: (0, 0)),
            pl.BlockSpec((1, 3 * d), lambda i: (0, 0)),
        ],
        out_specs=pl.BlockSpec((_BLK, _HOPS, d), lambda i: (i, 0, 0)),
        out_shape=jax.ShapeDtypeStruct((B, _HOPS, d), jnp.float32),
    )(query, relation_weight, eos_weight, w_ih, w_hh,
      b_ih.reshape(1, -1), b_hh.reshape(1, -1))
    masks = jnp.ones((B, _HOPS), dtype=bool)
    return subgoals, masks


# speedup vs baseline: 2.2658x; 1.0316x over previous
import jax
import jax.numpy as jnp
from jax.experimental import pallas as pl

_D = 256
_K = 1024
_HOPS = 8
_BLK = 512


def _mm(a, b_t):
    return jax.lax.dot_general(
        a, b_t, (((1,), (1,)), ((), ())),
        preferred_element_type=jnp.float32)


def _rulernn_body(q_ref, rw_ref, eos_ref, wih_ref, whh_ref,
                  bih_ref, bhh_ref, out_ref):
    f32 = jnp.float32
    rw = rw_ref[...]
    wih = wih_ref[...]
    whh = whh_ref[...]
    bih = bih_ref[...]
    bhh = bhh_ref[...]

    wsq = jnp.sum(rw * rw, axis=1, keepdims=True).reshape(1, _K)

    def gru(gi, gh, h):
        r = jax.nn.sigmoid(gi[:, :_D] + gh[:, :_D])
        z = jax.nn.sigmoid(gi[:, _D:2 * _D] + gh[:, _D:2 * _D])
        n = jnp.tanh(gi[:, 2 * _D:] + r * gh[:, 2 * _D:])
        return (1.0 - z) * n + z * h

    gi2 = _mm(eos_ref[...], wih) + bih

    half = q_ref.shape[0] // 2
    iota_k = jax.lax.broadcasted_iota(jnp.int32, (half, _K), 1)

    def enc(q):
        gi = _mm(q, wih) + bih
        r = jax.nn.sigmoid(gi[:, :_D] + bhh[:, :_D])
        z = jax.nn.sigmoid(gi[:, _D:2 * _D] + bhh[:, _D:2 * _D])
        n = jnp.tanh(gi[:, 2 * _D:] + r * bhh[:, 2 * _D:])
        h = (1.0 - z) * n
        return gru(gi2, _mm(h, whh) + bhh, h)

    def select(h):
        hsq = jnp.sum(h * h, axis=1, keepdims=True)
        dist = hsq - 2.0 * _mm(h, rw) + wsq
        s = jnp.sqrt(jnp.maximum(dist, 1e-12))
        m = jnp.min(s, axis=1, keepdims=True)
        idx = jnp.min(jnp.where(s == m, iota_k, _K), axis=1, keepdims=True)
        onehot = (iota_k == idx).astype(f32)
        sg = jnp.dot(onehot, rw, preferred_element_type=f32)
        return sg

    def update(h, sg):
        gi = _mm(sg, wih) + bih
        gh = _mm(h, whh) + bhh
        return gru(gi, gh, h)

    hs = [enc(q_ref[c * half:(c + 1) * half, :]) for c in range(2)]
    for hop in range(_HOPS):
        picks = [select(h) for h in hs]
        for c, sg in enumerate(picks):
            out_ref[c * half:(c + 1) * half, hop, :] = sg
        hs = [update(h, sg) for h, sg in zip(hs, picks)]


def kernel(query, relation_weight, eos_weight, w_ih, w_hh, b_ih, b_hh):
    B, d = query.shape
    subgoals = pl.pallas_call(
        _rulernn_body,
        grid=(B // _BLK,),
        in_specs=[
            pl.BlockSpec((_BLK, d), lambda i: (i, 0)),
            pl.BlockSpec((_K, d), lambda i: (0, 0)),
            pl.BlockSpec((1, d), lambda i: (0, 0)),
            pl.BlockSpec((3 * d, d), lambda i: (0, 0)),
            pl.BlockSpec((3 * d, d), lambda i: (0, 0)),
            pl.BlockSpec((1, 3 * d), lambda i: (0, 0)),
            pl.BlockSpec((1, 3 * d), lambda i: (0, 0)),
        ],
        out_specs=pl.BlockSpec((_BLK, _HOPS, d), lambda i: (i, 0, 0)),
        out_shape=jax.ShapeDtypeStruct((B, _HOPS, d), jnp.float32),
    )(query, relation_weight, eos_weight, w_ih, w_hh,
      b_ih.reshape(1, -1), b_hh.reshape(1, -1))
    masks = jnp.ones((B, _HOPS), dtype=bool)
    return subgoals, masks


# speedup vs baseline: 2.3255x; 1.0263x over previous
import jax
import jax.numpy as jnp
from jax.experimental import pallas as pl

_D = 256
_K = 1024
_HOPS = 8
_BLK = 512


def _mm(a, b_t):
    return jax.lax.dot_general(
        a, b_t, (((1,), (1,)), ((), ())),
        preferred_element_type=jnp.float32)


def _rulernn_body(q_ref, rw_ref, eos_ref, wih_ref, whh_ref,
                  bih_ref, bhh_ref, out_ref):
    f32 = jnp.float32
    rw = rw_ref[...]
    wih = wih_ref[...]
    whh = whh_ref[...]
    bih = bih_ref[...]
    bhh = bhh_ref[...]

    wsq = jnp.sum(rw * rw, axis=1, keepdims=True).reshape(1, _K)
    rw_m2 = rw * -2.0

    def gru(gi, gh, h):
        r = jax.nn.sigmoid(gi[:, :_D] + gh[:, :_D])
        z = jax.nn.sigmoid(gi[:, _D:2 * _D] + gh[:, _D:2 * _D])
        n = jnp.tanh(gi[:, 2 * _D:] + r * gh[:, 2 * _D:])
        return (1.0 - z) * n + z * h

    gi2 = _mm(eos_ref[...], wih) + bih

    half = q_ref.shape[0] // 2
    iota_k = jax.lax.broadcasted_iota(jnp.int32, (half, _K), 1)

    def enc(q):
        gi = _mm(q, wih) + bih
        r = jax.nn.sigmoid(gi[:, :_D] + bhh[:, :_D])
        z = jax.nn.sigmoid(gi[:, _D:2 * _D] + bhh[:, _D:2 * _D])
        n = jnp.tanh(gi[:, 2 * _D:] + r * bhh[:, 2 * _D:])
        h = (1.0 - z) * n
        return gru(gi2, _mm(h, whh) + bhh, h)

    def select(h):
        hsq = jnp.sum(h * h, axis=1, keepdims=True)
        dist = hsq + _mm(h, rw_m2) + wsq
        s = jnp.sqrt(jnp.maximum(dist, 1e-12))
        m = jnp.min(s, axis=1, keepdims=True)
        idx = jnp.min(jnp.where(s == m, iota_k, _K), axis=1, keepdims=True)
        onehot = (iota_k == idx).astype(f32)
        sg = jnp.dot(onehot, rw, preferred_element_type=f32)
        return sg

    def update(h, sg):
        gi = _mm(sg, wih) + bih
        gh = _mm(h, whh) + bhh
        return gru(gi, gh, h)

    hs = [enc(q_ref[c * half:(c + 1) * half, :]) for c in range(2)]
    for hop in range(_HOPS):
        picks = [select(h) for h in hs]
        for c, sg in enumerate(picks):
            out_ref[c * half:(c + 1) * half, hop, :] = sg
        hs = [update(h, sg) for h, sg in zip(hs, picks)]


def kernel(query, relation_weight, eos_weight, w_ih, w_hh, b_ih, b_hh):
    B, d = query.shape
    subgoals = pl.pallas_call(
        _rulernn_body,
        grid=(B // _BLK,),
        in_specs=[
            pl.BlockSpec((_BLK, d), lambda i: (i, 0)),
            pl.BlockSpec((_K, d), lambda i: (0, 0)),
            pl.BlockSpec((1, d), lambda i: (0, 0)),
            pl.BlockSpec((3 * d, d), lambda i: (0, 0)),
            pl.BlockSpec((3 * d, d), lambda i: (0, 0)),
            pl.BlockSpec((1, 3 * d), lambda i: (0, 0)),
            pl.BlockSpec((1, 3 * d), lambda i: (0, 0)),
        ],
        out_specs=pl.BlockSpec((_BLK, _HOPS, d), lambda i: (i, 0, 0)),
        out_shape=jax.ShapeDtypeStruct((B, _HOPS, d), jnp.float32),
    )(query, relation_weight, eos_weight, w_ih, w_hh,
      b_ih.reshape(1, -1), b_hh.reshape(1, -1))
    masks = jnp.ones((B, _HOPS), dtype=bool)
    return subgoals, masks


# speedup vs baseline: 2.5248x; 1.0857x over previous
import jax
import jax.numpy as jnp
from jax.experimental import pallas as pl

_D = 256
_K = 1024
_HOPS = 8
_BLK = 1024


def _mm(a, b_t):
    return jax.lax.dot_general(
        a, b_t, (((1,), (1,)), ((), ())),
        preferred_element_type=jnp.float32)


def _rulernn_body(q_ref, rw_ref, eos_ref, wih_ref, whh_ref,
                  bih_ref, bhh_ref, out_ref):
    f32 = jnp.float32
    rw = rw_ref[...]
    wih = wih_ref[...]
    whh = whh_ref[...]
    bih = bih_ref[...]
    bhh = bhh_ref[...]

    wsq = jnp.sum(rw * rw, axis=1, keepdims=True).reshape(1, _K)
    rw_m2 = rw * -2.0

    def gru(gi, gh, h):
        r = jax.nn.sigmoid(gi[:, :_D] + gh[:, :_D])
        z = jax.nn.sigmoid(gi[:, _D:2 * _D] + gh[:, _D:2 * _D])
        n = jnp.tanh(gi[:, 2 * _D:] + r * gh[:, 2 * _D:])
        return (1.0 - z) * n + z * h

    gi2 = _mm(eos_ref[...], wih) + bih

    half = q_ref.shape[0] // 2
    iota_k = jax.lax.broadcasted_iota(jnp.int32, (half, _K), 1)

    def enc(q):
        gi = _mm(q, wih) + bih
        r = jax.nn.sigmoid(gi[:, :_D] + bhh[:, :_D])
        z = jax.nn.sigmoid(gi[:, _D:2 * _D] + bhh[:, _D:2 * _D])
        n = jnp.tanh(gi[:, 2 * _D:] + r * bhh[:, 2 * _D:])
        h = (1.0 - z) * n
        return gru(gi2, _mm(h, whh) + bhh, h)

    def select(h):
        hsq = jnp.sum(h * h, axis=1, keepdims=True)
        dist = hsq + _mm(h, rw_m2) + wsq
        s = jnp.sqrt(jnp.maximum(dist, 1e-12))
        m = jnp.min(s, axis=1, keepdims=True)
        idx = jnp.min(jnp.where(s == m, iota_k, _K), axis=1, keepdims=True)
        onehot = (iota_k == idx).astype(f32)
        sg = jnp.dot(onehot, rw, preferred_element_type=f32)
        return sg

    def update(h, sg):
        gi = _mm(sg, wih) + bih
        gh = _mm(h, whh) + bhh
        return gru(gi, gh, h)

    hs = [enc(q_ref[c * half:(c + 1) * half, :]) for c in range(2)]
    for hop in range(_HOPS):
        picks = [select(h) for h in hs]
        for c, sg in enumerate(picks):
            out_ref[c * half:(c + 1) * half, hop, :] = sg
        hs = [update(h, sg) for h, sg in zip(hs, picks)]


def kernel(query, relation_weight, eos_weight, w_ih, w_hh, b_ih, b_hh):
    B, d = query.shape
    subgoals = pl.pallas_call(
        _rulernn_body,
        grid=(B // _BLK,),
        in_specs=[
            pl.BlockSpec((_BLK, d), lambda i: (i, 0)),
            pl.BlockSpec((_K, d), lambda i: (0, 0)),
            pl.BlockSpec((1, d), lambda i: (0, 0)),
            pl.BlockSpec((3 * d, d), lambda i: (0, 0)),
            pl.BlockSpec((3 * d, d), lambda i: (0, 0)),
            pl.BlockSpec((1, 3 * d), lambda i: (0, 0)),
            pl.BlockSpec((1, 3 * d), lambda i: (0, 0)),
        ],
        out_specs=pl.BlockSpec((_BLK, _HOPS, d), lambda i: (i, 0, 0)),
        out_shape=jax.ShapeDtypeStruct((B, _HOPS, d), jnp.float32),
    )(query, relation_weight, eos_weight, w_ih, w_hh,
      b_ih.reshape(1, -1), b_hh.reshape(1, -1))
    masks = jnp.ones((B, _HOPS), dtype=bool)
    return subgoals, masks
